# baseline (speedup 1.0000x reference)
import functools

import jax
import jax.numpy as jnp
from jax import lax
from jax.experimental import pallas as pl
from jax.experimental.pallas import tpu as pltpu
from jax.experimental.pallas import tpu_sc as plsc

EMBED = 64
BATCH = 16384
LABELS = 50
VOCAB = 1000000

NC, NS = 2, 16
NW = NC * NS
BB = 128
NB = BATCH // BB
BPW = NB // NW
LANES = 16

TCB = 2048
NBLK = 245
HALF = TCB * NBLK
WTBLK = VOCAB // TCB


def _tc_repack(wt):
    def body(xa_ref, xb_ref, o_ref):
        o_ref[:, 0:EMBED] = jnp.transpose(xa_ref[...])
        o_ref[:, EMBED:2 * EMBED] = jnp.transpose(xb_ref[...])

    return pl.pallas_call(
        body,
        grid=(NBLK,),
        in_specs=[
            pl.BlockSpec((EMBED, TCB), lambda i: (0, i)),
            pl.BlockSpec((EMBED, TCB), lambda i: (0, jnp.minimum(i + NBLK, WTBLK))),
        ],
        out_specs=pl.BlockSpec((TCB, 2 * EMBED), lambda i: (i, 0)),
        out_shape=jax.ShapeDtypeStruct((HALF, 2 * EMBED), jnp.float32),
    )(wt, wt)


def _sc_embed_sum(p_tab, idx_t):
    mesh = plsc.VectorSubcoreMesh(core_axis_name="c", subcore_axis_name="s")

    @functools.partial(
        pl.kernel,
        out_type=jax.ShapeDtypeStruct((BATCH // 2, 2 * EMBED), jnp.float32),
        mesh=mesh,
        compiler_params=pltpu.CompilerParams(use_tc_tiling_on_sc=True,
                                             needs_layout_passes=False),
        scratch_types=[
            pltpu.VMEM((LABELS, BB), jnp.int32),
            pltpu.VMEM((BB,), jnp.int32),
            pltpu.VMEM((BB,), jnp.int32),
            pltpu.VMEM((BB,), jnp.int32),
            pltpu.VMEM((BB,), jnp.int32),
            pltpu.VMEM((BB // 2, 2 * EMBED), jnp.float32),
            pltpu.VMEM((BB, 2 * EMBED), jnp.float32),
            pltpu.VMEM((BB, 2 * EMBED), jnp.float32),
            pltpu.SemaphoreType.DMA,
            pltpu.SemaphoreType.DMA,
        ],
    )
    def k(p_hbm, idx_hbm, out_hbm, idx_v, col0, col1, h0, h1, acc,
          buf0, buf1, sem0, sem1):
        wid = lax.axis_index("s") * NC + lax.axis_index("c")
        bufs = (buf0, buf1)
        cols = (col0, col1)
        hbufs = (h0, h1)
        sems = (sem0, sem1)
        zero = jnp.zeros((LANES,), jnp.float32)
        lane_iota = lax.iota(jnp.int32, LANES)

        def build_col(l, b):
            for c in range(BB // LANES):
                sl = pl.ds(c * LANES, LANES)
                v = idx_v[l, sl]
                hi = v >= HALF
                cols[b][sl] = v - jnp.where(hi, HALF, 0)
                hbufs[b][sl] = jnp.where(hi, EMBED, 0)

        def gather_start(b):
            pltpu.make_async_copy(p_hbm.at[cols[b]], bufs[b], sems[b]).start()

        def gather_wait(b):
            pltpu.make_async_copy(p_hbm.at[cols[b]], bufs[b], sems[b]).wait()

        def accum(b):
            buf = bufs[b]
            hb = hbufs[b]

            def gbody(g, _):
                r0 = g * LANES
                riota = lane_iota + r0
                qv = lax.shift_right_logical(riota, 1)
                src_col = hb[pl.ds(r0, LANES)]
                dst_col = lax.mul(lax.rem(riota, 2), EMBED)
                one = jnp.full((LANES,), 1, jnp.int32)
                for _e in range(EMBED):
                    vals = plsc.load_gather(buf, [riota, src_col])
                    plsc.addupdate_scatter(acc, [qv, dst_col], vals)
                    src_col = src_col + one
                    dst_col = dst_col + one
                return 0

            lax.fori_loop(0, BB // LANES, gbody, 0)

        def block_body(kblk, _):
            jb = wid * BPW + kblk
            b0 = jb * BB
            pltpu.sync_copy(idx_hbm.at[:, pl.ds(b0, BB)], idx_v)

            def zbody(i, _):
                for c in range(2 * EMBED // LANES):
                    acc[i, pl.ds(c * LANES, LANES)] = zero
                return 0

            lax.fori_loop(0, BB // 2, zbody, 0)

            build_col(0, 0)
            gather_start(0)
            build_col(1, 1)
            gather_start(1)

            def pair_body(i, _):
                for b in range(2):
                    l = 2 * i + b
                    gather_wait(b)
                    accum(b)
                    build_col(l + 2, b)
                    gather_start(b)
                return 0

            lax.fori_loop(0, LABELS // 2 - 1, pair_body, 0)

            for b in range(2):
                gather_wait(b)
                accum(b)

            pltpu.sync_copy(acc, out_hbm.at[pl.ds(jb * (BB // 2), BB // 2)])
            return 0

        lax.fori_loop(0, BPW, block_body, 0)

    return k(p_tab, idx_t)


def kernel(inputs, weight):
    wt = weight.T
    p_tab = _tc_repack(wt)
    idx_t = inputs.astype(jnp.int32).T
    out2 = _sc_embed_sum(p_tab, idx_t)
    return out2.reshape(BATCH, EMBED)


# speedup vs baseline: 3.2818x; 3.2818x over previous
import functools

import jax
import jax.numpy as jnp
from jax import lax
from jax.experimental import pallas as pl
from jax.experimental.pallas import tpu as pltpu
from jax.experimental.pallas import tpu_sc as plsc

EMBED = 64
BATCH = 16384
LABELS = 50
VOCAB = 1000000

NC, NS = 2, 16
NW = NC * NS
BB = 128
NB = BATCH // BB
BPW = NB // NW
LANES = 16

TCB = 2048
NBLK = 245
HALF = TCB * NBLK
WTBLK = VOCAB // TCB


def _tc_repack(wt):
    def body(xa_ref, xb_ref, o_ref):
        o_ref[:, 0:EMBED] = jnp.transpose(xa_ref[...])
        o_ref[:, EMBED:2 * EMBED] = jnp.transpose(xb_ref[...])

    return pl.pallas_call(
        body,
        grid=(NBLK,),
        in_specs=[
            pl.BlockSpec((EMBED, TCB), lambda i: (0, i)),
            pl.BlockSpec((EMBED, TCB), lambda i: (0, jnp.minimum(i + NBLK, WTBLK))),
        ],
        out_specs=pl.BlockSpec((TCB, 2 * EMBED), lambda i: (i, 0)),
        out_shape=jax.ShapeDtypeStruct((HALF, 2 * EMBED), jnp.float32),
    )(wt, wt)


def _sc_embed_sum(p_tab, idx_t):
    mesh = plsc.VectorSubcoreMesh(core_axis_name="c", subcore_axis_name="s")

    @functools.partial(
        pl.kernel,
        out_type=jax.ShapeDtypeStruct((BATCH // 2, 2 * EMBED), jnp.float32),
        mesh=mesh,
        compiler_params=pltpu.CompilerParams(use_tc_tiling_on_sc=True,
                                             needs_layout_passes=False),
        scratch_types=[
            pltpu.VMEM((LABELS, BB), jnp.int32),
            pltpu.VMEM((BB,), jnp.int32),
            pltpu.VMEM((BB,), jnp.int32),
            pltpu.VMEM((BB,), jnp.int32),
            pltpu.VMEM((BB,), jnp.int32),
            pltpu.VMEM((BB // 2, 2 * EMBED), jnp.float32),
            pltpu.VMEM((BB, 2 * EMBED), jnp.float32),
            pltpu.VMEM((BB, 2 * EMBED), jnp.float32),
            pltpu.SemaphoreType.DMA,
            pltpu.SemaphoreType.DMA,
        ],
    )
    def k(p_hbm, idx_hbm, out_hbm, idx_v, col0, col1, h0, h1, acc,
          buf0, buf1, sem0, sem1):
        wid = lax.axis_index("s") * NC + lax.axis_index("c")
        bufs = (buf0, buf1)
        cols = (col0, col1)
        hbufs = (h0, h1)
        sems = (sem0, sem1)
        zero = jnp.zeros((LANES,), jnp.float32)
        lane_iota = lax.iota(jnp.int32, LANES)

        def build_col(l, b):
            for c in range(BB // LANES):
                sl = pl.ds(c * LANES, LANES)
                v = idx_v[l, sl]
                hi = v >= HALF
                cols[b][sl] = v - jnp.where(hi, HALF, 0)
                hbufs[b][sl] = jnp.where(hi, EMBED, 0)

        def gather_start(b):
            pltpu.make_async_copy(p_hbm.at[cols[b]], bufs[b], sems[b]).start()

        def gather_wait(b):
            pltpu.make_async_copy(p_hbm.at[cols[b]], bufs[b], sems[b]).wait()

        def accum(b):
            buf = bufs[b]
            hb = hbufs[b]

            def gbody(g, _):
                r0 = g * LANES
                hv = hb[pl.ds(r0, LANES)]
                for kk in range(LANES):
                    r = r0 + kk
                    h = hv[kk]
                    q = lax.shift_right_logical(r, 1)
                    dc = lax.mul(lax.rem(r, 2), EMBED)
                    for c in range(EMBED // LANES):
                        vals = buf[r, pl.ds(h + c * LANES, LANES)]
                        plsc.addupdate(
                            acc.at[q, pl.ds(dc + c * LANES, LANES)], vals)
                return 0

            lax.fori_loop(0, BB // LANES, gbody, 0)

        def block_body(kblk, _):
            jb = wid * BPW + kblk
            b0 = jb * BB
            pltpu.sync_copy(idx_hbm.at[:, pl.ds(b0, BB)], idx_v)

            def zbody(i, _):
                for c in range(2 * EMBED // LANES):
                    acc[i, pl.ds(c * LANES, LANES)] = zero
                return 0

            lax.fori_loop(0, BB // 2, zbody, 0)

            build_col(0, 0)
            gather_start(0)
            build_col(1, 1)
            gather_start(1)

            def pair_body(i, _):
                for b in range(2):
                    l = 2 * i + b
                    gather_wait(b)
                    accum(b)
                    build_col(l + 2, b)
                    gather_start(b)
                return 0

            lax.fori_loop(0, LABELS // 2 - 1, pair_body, 0)

            for b in range(2):
                gather_wait(b)
                accum(b)

            pltpu.sync_copy(acc, out_hbm.at[pl.ds(jb * (BB // 2), BB // 2)])
            return 0

        lax.fori_loop(0, BPW, block_body, 0)

    return k(p_tab, idx_t)


def kernel(inputs, weight):
    wt = weight.T
    p_tab = _tc_repack(wt)
    idx_t = inputs.astype(jnp.int32).T
    out2 = _sc_embed_sum(p_tab, idx_t)
    return out2.reshape(BATCH, EMBED)


# speedup vs baseline: 3.3495x; 1.0206x over previous
import functools

import jax
import jax.numpy as jnp
from jax import lax
from jax.experimental import pallas as pl
from jax.experimental.pallas import tpu as pltpu
from jax.experimental.pallas import tpu_sc as plsc

EMBED = 64
BATCH = 16384
LABELS = 50
VOCAB = 1000000

NC, NS = 2, 16
NW = NC * NS
BB = 128
NB = BATCH // BB
BPW = NB // NW
LANES = 16
NBUF = 3

TCB = 4096
TCGRID = (VOCAB + TCB - 1) // TCB


def _tc_repack(wt):
    def body(x_ref, o_ref):
        ident = (lax.broadcasted_iota(jnp.int32, (EMBED, EMBED), 0) ==
                 lax.broadcasted_iota(jnp.int32, (EMBED, EMBED), 1)
                 ).astype(jnp.float32)
        o_ref[:, 0:EMBED] = jax.lax.dot_general(
            x_ref[...], ident, (((0,), (0,)), ((), ())),
            preferred_element_type=jnp.float32)

    return pl.pallas_call(
        body,
        grid=(TCGRID,),
        in_specs=[pl.BlockSpec((EMBED, TCB), lambda i: (0, i))],
        out_specs=pl.BlockSpec((TCB, 2 * EMBED), lambda i: (i, 0)),
        out_shape=jax.ShapeDtypeStruct((VOCAB, 2 * EMBED), jnp.float32),
    )(wt)


def _sc_embed_sum(p_tab, idx_t):
    mesh = plsc.VectorSubcoreMesh(core_axis_name="c", subcore_axis_name="s")

    @functools.partial(
        pl.kernel,
        out_type=jax.ShapeDtypeStruct((BATCH // 2, 2 * EMBED), jnp.float32),
        mesh=mesh,
        compiler_params=pltpu.CompilerParams(use_tc_tiling_on_sc=True),
        scratch_types=[
            pltpu.VMEM((LABELS, BB), jnp.int32),
            pltpu.VMEM((BB // 2, 2 * EMBED), jnp.float32),
            pltpu.VMEM((BB, 2 * EMBED), jnp.float32),
            pltpu.VMEM((BB, 2 * EMBED), jnp.float32),
            pltpu.VMEM((BB, 2 * EMBED), jnp.float32),
            pltpu.SemaphoreType.DMA,
            pltpu.SemaphoreType.DMA,
            pltpu.SemaphoreType.DMA,
        ],
    )
    def k(p_hbm, idx_hbm, out_hbm, idx_v, acc,
          buf0, buf1, buf2, sem0, sem1, sem2):
        wid = lax.axis_index("s") * NC + lax.axis_index("c")
        bufs = (buf0, buf1, buf2)
        sems = (sem0, sem1, sem2)
        zero = jnp.zeros((LANES,), jnp.float32)

        def gather_start(l, b):
            pltpu.make_async_copy(
                p_hbm.at[idx_v.at[l]], bufs[b], sems[b]).start()

        def gather_wait(b):
            pltpu.make_async_copy(
                p_hbm.at[idx_v.at[0]], bufs[b], sems[b]).wait()

        def accum(b):
            buf = bufs[b]

            def qbody(i, _):
                for qq in range(2):
                    q = i * 2 + qq
                    for kk in range(2):
                        r = q * 2 + kk
                        for c in range(EMBED // LANES):
                            sl = pl.ds(c * LANES, LANES)
                            dsl = pl.ds(kk * EMBED + c * LANES, LANES)
                            plsc.addupdate(acc.at[q, dsl], buf[r, sl])
                return 0

            lax.fori_loop(0, BB // 4, qbody, 0)

        def block_body(kblk, _):
            jb = wid * BPW + kblk
            b0 = jb * BB
            pltpu.sync_copy(idx_hbm.at[:, pl.ds(b0, BB)], idx_v)

            def zbody(i, _):
                for qq in range(4):
                    q = i * 4 + qq
                    for c in range(2 * EMBED // LANES):
                        acc[q, pl.ds(c * LANES, LANES)] = zero
                return 0

            lax.fori_loop(0, BB // 8, zbody, 0)

            for b in range(NBUF):
                gather_start(b, b)

            def tri_body(i, _):
                for b in range(NBUF):
                    l = NBUF * i + b
                    gather_wait(b)
                    accum(b)
                    gather_start(l + NBUF, b)
                return 0

            lax.fori_loop(0, 15, tri_body, 0)

            for b, l in ((0, 45), (1, 46), (2, 47), (0, 48), (1, 49)):
                gather_wait(b)
                accum(b)
                if l + NBUF <= LABELS - 1:
                    gather_start(l + NBUF, b)

            pltpu.sync_copy(acc, out_hbm.at[pl.ds(jb * (BB // 2), BB // 2)])
            return 0

        lax.fori_loop(0, BPW, block_body, 0)

    return k(p_tab, idx_t)


def kernel(inputs, weight):
    wt = weight.T
    p_tab = _tc_repack(wt)
    idx_t = inputs.astype(jnp.int32).T
    out2 = _sc_embed_sum(p_tab, idx_t)
    return out2.reshape(BATCH, EMBED)
